# baseline (speedup 1.0000x reference)
import functools

import jax
import jax.numpy as jnp
from jax import lax
from jax.experimental import pallas as pl
from jax.experimental.pallas import tpu as pltpu
from jax.experimental.pallas import tpu_sc as plsc

N = 320000
D = 128
S = 1024

NC = 2
NS = 16
NW = NC * NS
ROWS_PER_W = N // NW
XBLOCK = 400
NXCHUNK = ROWS_PER_W // XBLOCK
SCHUNK = 80
NSUB = XBLOCK // SCHUNK
NIDROW = ROWS_PER_W // SCHUNK
ROWS_PER_TILE_OUT = S // NS
ZROWS = 16


def _sc_body(x_hbm, batch_hbm, out_hbm, xb0, xb1, ibuf, zbuf, acc,
             sx0, sx1, ss0, ss1):
    xb = (xb0, xb1)
    sx = (sx0, sx1)
    ss = (ss0, ss1)

    c = lax.axis_index("c")
    s = lax.axis_index("s")
    wid = c * NS + s

    pltpu.async_copy(batch_hbm.at[wid], ibuf, sx[1])

    def zrow(i, _):
        for j in range(D // 16):
            zbuf[i, pl.ds(j * 16, 16)] = jnp.zeros((16,), jnp.float32)
        return 0
    lax.fori_loop(0, ZROWS, zrow, 0)
    for k in range(ROWS_PER_TILE_OUT // ZROWS):
        pltpu.sync_copy(
            zbuf, acc.at[pl.ds(s * ROWS_PER_TILE_OUT + k * ZROWS, ZROWS)])
    pltpu.make_async_copy(batch_hbm.at[wid], ibuf, sx[1]).wait()
    plsc.subcore_barrier()

    def start_load(ch, b):
        pltpu.async_copy(x_hbm.at[wid, ch], xb[b], sx[b])

    def wait_load(ch, b):
        pltpu.make_async_copy(x_hbm.at[wid, ch], xb[b], sx[b]).wait()

    def start_scatter(ch, b):
        for j in range(NSUB):
            pltpu.async_copy(
                xb[b].at[pl.ds(j * SCHUNK, SCHUNK)],
                acc.at[ibuf.at[ch * NSUB + j]], ss[b], add=True)

    def wait_scatter(ch, b):
        for j in range(NSUB):
            pltpu.make_async_copy(
                xb[b].at[pl.ds(j * SCHUNK, SCHUNK)],
                acc.at[ibuf.at[ch * NSUB + j]], ss[b]).wait()

    start_load(0, 0)

    def outer(k, _):
        for b in range(2):
            ch = 2 * k + b
            nb = 1 - b

            @pl.when(ch >= 1)
            def _():
                wait_scatter(ch - 1, nb)
            start_load(ch + 1, nb)
            wait_load(ch, b)
            start_scatter(ch, b)
        return 0

    lax.fori_loop(0, (NXCHUNK - 1) // 2, outer, 0)

    wait_scatter(NXCHUNK - 2, 1)
    wait_load(NXCHUNK - 1, 0)
    start_scatter(NXCHUNK - 1, 0)
    wait_scatter(NXCHUNK - 1, 0)

    plsc.subcore_barrier()
    pltpu.sync_copy(
        acc.at[pl.ds(s * ROWS_PER_TILE_OUT, ROWS_PER_TILE_OUT)],
        out_hbm.at[c, pl.ds(s * ROWS_PER_TILE_OUT, ROWS_PER_TILE_OUT)],
    )


_sc_stage = functools.partial(
    pl.kernel,
    out_type=jax.ShapeDtypeStruct((NC, S, D), jnp.float32),
    mesh=plsc.VectorSubcoreMesh(core_axis_name="c", subcore_axis_name="s"),
    scratch_types=[
        pltpu.VMEM((XBLOCK, D), jnp.float32),
        pltpu.VMEM((XBLOCK, D), jnp.float32),
        pltpu.VMEM((NIDROW, SCHUNK), jnp.int32),
        pltpu.VMEM((ZROWS, D), jnp.float32),
        pltpu.VMEM_SHARED((S, D), jnp.float32),
        pltpu.SemaphoreType.DMA,
        pltpu.SemaphoreType.DMA,
        pltpu.SemaphoreType.DMA,
        pltpu.SemaphoreType.DMA,
    ],
)(_sc_body)


def _add_body(p_ref, o_ref):
    o_ref[...] = p_ref[0] + p_ref[1]


def kernel(x, batch):
    xr = x.reshape(NW, NXCHUNK, XBLOCK, D)
    br = batch.reshape(NW, NIDROW, SCHUNK)
    partials = _sc_stage(xr, br)
    out = pl.pallas_call(
        _add_body,
        out_shape=jax.ShapeDtypeStruct((S, D), jnp.float32),
    )(partials)
    return out


# speedup vs baseline: 1.2582x; 1.2582x over previous
import functools

import jax
import jax.numpy as jnp
from jax import lax
from jax.experimental import pallas as pl
from jax.experimental.pallas import tpu as pltpu
from jax.experimental.pallas import tpu_sc as plsc

N = 320000
D = 128
S = 1024

NC = 2
NS = 16
NW = NC * NS
ROWS_PER_W = N // NW
CHUNK = 80
NCHUNK = ROWS_PER_W // CHUNK
NBUF = 4
ROWS_PER_TILE_OUT = S // NS
ZROWS = 16


def _sc_body(x_hbm, batch_hbm, out_hbm, xb0, xb1, xb2, xb3, ibuf, zbuf, acc,
             sx0, sx1, sx2, sx3, ss0, ss1, ss2, ss3):
    xb = (xb0, xb1, xb2, xb3)
    sx = (sx0, sx1, sx2, sx3)
    ss = (ss0, ss1, ss2, ss3)

    c = lax.axis_index("c")
    s = lax.axis_index("s")
    wid = c * NS + s

    pltpu.async_copy(batch_hbm.at[wid], ibuf, sx[NBUF - 1])

    def zrow(i, _):
        for j in range(D // 16):
            zbuf[i, pl.ds(j * 16, 16)] = jnp.zeros((16,), jnp.float32)
        return 0
    lax.fori_loop(0, ZROWS, zrow, 0)
    for k in range(ROWS_PER_TILE_OUT // ZROWS):
        pltpu.sync_copy(
            zbuf, acc.at[pl.ds(s * ROWS_PER_TILE_OUT + k * ZROWS, ZROWS)])
    pltpu.make_async_copy(batch_hbm.at[wid], ibuf, sx[NBUF - 1]).wait()
    plsc.subcore_barrier()

    def start_load(ch, b):
        pltpu.async_copy(x_hbm.at[wid, ch], xb[b], sx[b])

    def wait_load(ch, b):
        pltpu.make_async_copy(x_hbm.at[wid, ch], xb[b], sx[b]).wait()

    def start_scatter(ch, b):
        pltpu.async_copy(xb[b], acc.at[ibuf.at[ch]], ss[b], add=True)

    def wait_scatter(ch, b):
        pltpu.make_async_copy(xb[b], acc.at[ibuf.at[ch]], ss[b]).wait()

    for p in range(NBUF - 1):
        start_load(p, p)

    def outer(k, _):
        for b in range(NBUF):
            ch = NBUF * k + b
            wait_load(ch, b)
            start_scatter(ch, b)

            @pl.when(ch >= 1)
            def _():
                wait_scatter(ch - 1, (b - 1) % NBUF)

            @pl.when(ch + NBUF - 1 < NCHUNK)
            def _():
                start_load(ch + NBUF - 1, (b - 1) % NBUF)
        return 0

    lax.fori_loop(0, (NCHUNK - 1) // NBUF, outer, 0)

    wait_load(NCHUNK - 1, 0)
    start_scatter(NCHUNK - 1, 0)
    wait_scatter(NCHUNK - 2, NBUF - 1)
    wait_scatter(NCHUNK - 1, 0)

    plsc.subcore_barrier()
    pltpu.sync_copy(
        acc.at[pl.ds(s * ROWS_PER_TILE_OUT, ROWS_PER_TILE_OUT)],
        out_hbm.at[c, pl.ds(s * ROWS_PER_TILE_OUT, ROWS_PER_TILE_OUT)],
    )


_sc_stage = functools.partial(
    pl.kernel,
    out_type=jax.ShapeDtypeStruct((NC, S, D), jnp.float32),
    mesh=plsc.VectorSubcoreMesh(core_axis_name="c", subcore_axis_name="s"),
    scratch_types=[
        pltpu.VMEM((CHUNK, D), jnp.float32),
        pltpu.VMEM((CHUNK, D), jnp.float32),
        pltpu.VMEM((CHUNK, D), jnp.float32),
        pltpu.VMEM((CHUNK, D), jnp.float32),
        pltpu.VMEM((NCHUNK, CHUNK), jnp.int32),
        pltpu.VMEM((ZROWS, D), jnp.float32),
        pltpu.VMEM_SHARED((S, D), jnp.float32),
        pltpu.SemaphoreType.DMA,
        pltpu.SemaphoreType.DMA,
        pltpu.SemaphoreType.DMA,
        pltpu.SemaphoreType.DMA,
        pltpu.SemaphoreType.DMA,
        pltpu.SemaphoreType.DMA,
        pltpu.SemaphoreType.DMA,
        pltpu.SemaphoreType.DMA,
    ],
)(_sc_body)


def _add_body(p_ref, o_ref):
    o_ref[...] = p_ref[0] + p_ref[1]


def kernel(x, batch):
    xr = x.reshape(NW, NCHUNK, CHUNK, D)
    br = batch.reshape(NW, NCHUNK, CHUNK)
    partials = _sc_stage(xr, br)
    out = pl.pallas_call(
        _add_body,
        out_shape=jax.ShapeDtypeStruct((S, D), jnp.float32),
    )(partials)
    return out
